# baseline (speedup 1.0000x reference)
import functools

import jax
import jax.numpy as jnp
from jax import lax
from jax.experimental import pallas as pl
from jax.experimental.pallas import tpu as pltpu
from jax.experimental.pallas import tpu_sc as plsc

_DIM = 1024
_BATCH = 32
_DBLK = 32
_NWORK = 4
_RPW = _BATCH // _NWORK


def _sc_gather_body(t_hbm, table_hbm, out_hbm, idx_v, rows_v, sem):
    wid = lax.axis_index("s") * 2 + lax.axis_index("c")

    @pl.when(wid < _NWORK)
    def _():
        base = pl.multiple_of(wid * _RPW, _RPW)
        pltpu.sync_copy(t_hbm.at[pl.ds(base, _RPW)], idx_v)
        pltpu.async_copy(table_hbm.at[idx_v], rows_v, sem).wait()
        pltpu.sync_copy(rows_v, out_hbm.at[pl.ds(base, _RPW)])


def _sc_gather(t, table):
    mesh = plsc.VectorSubcoreMesh(core_axis_name="c", subcore_axis_name="s")
    return pl.kernel(
        _sc_gather_body,
        mesh=mesh,
        out_type=jax.ShapeDtypeStruct((_BATCH, _DIM), jnp.float32),
        scratch_types=[
            pltpu.VMEM((_RPW,), jnp.int32),
            pltpu.VMEM((_RPW, _DIM), jnp.float32),
            pltpu.SemaphoreType.DMA,
        ],
    )(t, table)


def _tc_add_body(rows_ref, pos_ref, out_ref):
    pos_vals = pos_ref[0, :, 0, 0]
    out_ref[0] = pos_vals[:, None, None] + rows_ref[:, :][None, :, :]


def _tc_add(rows, pos_emb):
    return pl.pallas_call(
        _tc_add_body,
        grid=(_DIM // _DBLK,),
        in_specs=[
            pl.BlockSpec((_BATCH, _DIM), lambda i: (0, 0)),
            pl.BlockSpec((1, _DBLK, 1, 1), lambda i: (0, i, 0, 0)),
        ],
        out_specs=pl.BlockSpec((1, _DBLK, _BATCH, _DIM), lambda i: (0, i, 0, 0)),
        out_shape=jax.ShapeDtypeStruct((1, _DIM, _BATCH, _DIM), jnp.float32),
    )(rows, pos_emb)


def kernel(t, table, pos_emb):
    rows = _sc_gather(t.astype(jnp.int32), table)
    return _tc_add(rows, pos_emb)


# speedup vs baseline: 1.3890x; 1.3890x over previous
import jax
import jax.numpy as jnp
from jax.experimental import pallas as pl
from jax.experimental.pallas import tpu as pltpu

_DIM = 1024
_BATCH = 32
_DBLK = 32


def _tc_body(t_ref, table_ref, pos_ref, out_ref, rows_ref, sem):
    i = pl.program_id(0)

    @pl.when(i == 0)
    def _gather():
        copies = []
        for b in range(_BATCH):
            cp = pltpu.make_async_copy(
                table_ref.at[pl.ds(t_ref[b], 1), :],
                rows_ref.at[pl.ds(b, 1), :],
                sem,
            )
            cp.start()
            copies.append(cp)
        for cp in copies:
            cp.wait()

    pos_vals = pos_ref[0, :, 0, 0]
    out_ref[0] = pos_vals[:, None, None] + rows_ref[:, :][None, :, :]


def kernel(t, table, pos_emb):
    t = t.astype(jnp.int32)
    grid = (_DIM // _DBLK,)
    return pl.pallas_call(
        _tc_body,
        grid_spec=pltpu.PrefetchScalarGridSpec(
            num_scalar_prefetch=1,
            grid=grid,
            in_specs=[
                pl.BlockSpec(memory_space=pltpu.MemorySpace.HBM),
                pl.BlockSpec((1, _DBLK, 1, 1), lambda i, t_pref: (0, i, 0, 0)),
            ],
            out_specs=pl.BlockSpec(
                (1, _DBLK, _BATCH, _DIM), lambda i, t_pref: (0, i, 0, 0)
            ),
            scratch_shapes=[
                pltpu.VMEM((_BATCH, _DIM), jnp.float32),
                pltpu.SemaphoreType.DMA,
            ],
        ),
        out_shape=jax.ShapeDtypeStruct((1, _DIM, _BATCH, _DIM), jnp.float32),
    )(t, table, pos_emb)
